# Initial kernel scaffold; baseline (speedup 1.0000x reference)
import jax
import jax.numpy as jnp
from jax.experimental import pallas as pl


def kernel(input_embeddings, memories_labels, memories_embeddings, memories_weights):
    raise NotImplementedError("write your pallas kernel here")


# speedup vs baseline: 21.8240x; 21.8240x over previous
import dataclasses
import functools

import jax
import jax.numpy as jnp
from jax import lax
from jax.experimental import pallas as pl
from jax.experimental.pallas import tpu as pltpu
from jax.experimental.pallas import tpu_sc as plsc

NUM_CLASSES = 1000
MIN_W = 0.1
B = 4096
M = 200
L = 16
MP = 208
NCHUNK = MP // L
NW = 32
RPW = B // NW
CP = 1024

_mesh = plsc.VectorSubcoreMesh(core_axis_name="c", subcore_axis_name="s")

_cp = pltpu.CompilerParams()
if "needs_layout_passes" in pltpu.CompilerParams.__dataclass_fields__:
    _cp = dataclasses.replace(_cp, needs_layout_passes=False)


@jax.jit
def _sc_histogram(labels, weights, recip):
    @functools.partial(
        pl.kernel,
        mesh=_mesh,
        compiler_params=_cp,
        out_type=jax.ShapeDtypeStruct((B * NUM_CLASSES,), jnp.float32),
        scratch_types=[
            pltpu.VMEM((RPW * MP,), jnp.int32),
            pltpu.VMEM((RPW * MP,), jnp.float32),
            pltpu.VMEM((CP,), jnp.float32),
            pltpu.VMEM((256,), jnp.float32),
        ],
    )
    def k(lab_hbm, w_hbm, recip_hbm, out_hbm, lab_v, w_v, hist, recip_v):
        wid = lax.axis_index("s") * 2 + lax.axis_index("c")
        base = wid * RPW
        pltpu.sync_copy(lab_hbm.at[pl.ds(base * MP, RPW * MP)], lab_v)
        pltpu.sync_copy(w_hbm.at[pl.ds(base * MP, RPW * MP)], w_v)
        pltpu.sync_copy(recip_hbm, recip_v)

        zeros = jnp.zeros((L,), jnp.float32)

        @pl.loop(0, CP, step=L)
        def _(i):
            hist[pl.ds(i, L)] = zeros

        @pl.loop(0, RPW)
        def _(r):
            ws = [w_v[pl.ds(r * MP + c * L, L)] for c in range(NCHUNK)]
            masks = [w >= MIN_W for w in ws]
            cnt = jnp.zeros((L,), jnp.int32)
            for m in masks:
                cnt = cnt + jnp.where(m, 1, 0)
            total = jnp.sum(cnt)
            inv = plsc.load_gather(recip_v, [jnp.broadcast_to(total, (L,))])

            labs = [lab_v[pl.ds(r * MP + c * L, L)] for c in range(NCHUNK)]
            for c in range(NCHUNK):
                attn = ws[c] * jnp.where(masks[c], inv, 1e-10 * inv)
                plsc.addupdate_scatter(hist, [labs[c]], attn)

            pltpu.sync_copy(hist.at[pl.ds(0, NUM_CLASSES)],
                            out_hbm.at[pl.ds((base + r) * NUM_CLASSES,
                                             NUM_CLASSES)])

            for c in range(NCHUNK):
                plsc.store_scatter(hist, [labs[c]], zeros)

    return k(labels, weights, recip)


def kernel(input_embeddings, memories_labels, memories_embeddings,
           memories_weights):
    labels = jnp.pad(memories_labels.astype(jnp.int32), ((0, 0), (0, MP - M)))
    weights = jnp.pad(memories_weights, ((0, 0), (0, MP - M)))
    recip = 1.0 / jnp.maximum(jnp.arange(256, dtype=jnp.float32), 1.0)
    flat = _sc_histogram(labels.reshape(-1), weights.reshape(-1), recip)
    return flat.reshape(B, NUM_CLASSES)


# speedup vs baseline: 25.5027x; 1.1686x over previous
import dataclasses
import functools

import jax
import jax.numpy as jnp
from jax import lax
from jax.experimental import pallas as pl
from jax.experimental.pallas import tpu as pltpu
from jax.experimental.pallas import tpu_sc as plsc

NUM_CLASSES = 1000
MIN_W = 0.1
B = 4096
M = 200
L = 16
NFULL = 12
TAIL_OFF = 184
NW = 32
RPW = B // NW
RPB = 8
NBLK = RPW // RPB
SLAB = RPB * NUM_CLASSES

_mesh = plsc.VectorSubcoreMesh(core_axis_name="c", subcore_axis_name="s")

_cp = pltpu.CompilerParams()
if "needs_layout_passes" in pltpu.CompilerParams.__dataclass_fields__:
    _cp = dataclasses.replace(_cp, needs_layout_passes=False)


@jax.jit
def _sc_histogram(labels, weights, recip):
    @functools.partial(
        pl.kernel,
        mesh=_mesh,
        compiler_params=_cp,
        out_type=jax.ShapeDtypeStruct((B * NUM_CLASSES,), jnp.float32),
        scratch_types=[
            pltpu.VMEM((RPW, M), jnp.int32),
            pltpu.VMEM((RPW, M), jnp.float32),
            pltpu.VMEM((SLAB,), jnp.float32),
            pltpu.VMEM((SLAB,), jnp.float32),
            pltpu.VMEM((256,), jnp.float32),
            pltpu.SemaphoreType.DMA,
            pltpu.SemaphoreType.DMA,
        ],
    )
    def k(lab_hbm, w_hbm, recip_hbm, out_hbm, lab_v, w_v, slab0, slab1,
          recip_v, sem0, sem1):
        wid = lax.axis_index("s") * 2 + lax.axis_index("c")
        base = wid * RPW
        pltpu.sync_copy(lab_hbm.at[pl.ds(base, RPW)], lab_v)
        pltpu.sync_copy(w_hbm.at[pl.ds(base, RPW)], w_v)
        pltpu.sync_copy(recip_hbm, recip_v)

        slabs = (slab0, slab1)
        sems = (sem0, sem1)
        zeros = jnp.zeros((L,), jnp.float32)
        tail_mask = lax.iota(jnp.int32, L) >= (NFULL * L - TAIL_OFF)

        def zero_slab(slab):
            @pl.loop(0, SLAB, step=8 * L)
            def _(i):
                for u in range(8):
                    slab[pl.ds(i + u * L, L)] = zeros

        zero_slab(slab0)
        zero_slab(slab1)

        def do_row(slab, r, s):
            ws = [w_v[r, pl.ds(c * L, L)] for c in range(NFULL)]
            wt = w_v[r, pl.ds(TAIL_OFF, L)]
            masks = [w >= MIN_W for w in ws]
            mt = (wt >= MIN_W) & tail_mask
            cnt = plsc.all_reduce_population_count(mt)
            for m in masks:
                cnt = cnt + plsc.all_reduce_population_count(m)
            inv = plsc.load_gather(recip_v, [cnt])
            tiny_inv = 1e-10 * inv

            off = s * NUM_CLASSES
            for c in range(NFULL):
                lab = lab_v[r, pl.ds(c * L, L)] + off
                attn = ws[c] * jnp.where(masks[c], inv, tiny_inv)
                plsc.addupdate_scatter(slab, [lab], attn)
            labt = lab_v[r, pl.ds(TAIL_OFF, L)] + off
            attnt = wt * jnp.where(mt, inv, tiny_inv)
            plsc.addupdate_scatter(slab, [labt], attnt, mask=tail_mask)

        @pl.loop(0, NBLK, step=2)
        def _(rb):
            for j in range(2):
                rbx = rb + j
                slab, sem = slabs[j], sems[j]

                @pl.when(rbx >= 2)
                def _():
                    dst = out_hbm.at[
                        pl.ds((base + (rbx - 2) * RPB) * NUM_CLASSES, SLAB)]
                    pltpu.make_async_copy(slab, dst, sem).wait()
                    zero_slab(slab)

                for s in range(RPB):
                    do_row(slab, rbx * RPB + s, s)

                dst = out_hbm.at[
                    pl.ds((base + rbx * RPB) * NUM_CLASSES, SLAB)]
                pltpu.make_async_copy(slab, dst, sem).start()

        for j in range(2):
            rbx = NBLK - 2 + j
            dst = out_hbm.at[pl.ds((base + rbx * RPB) * NUM_CLASSES, SLAB)]
            pltpu.make_async_copy(slabs[j], dst, sems[j]).wait()

    return k(labels, weights, recip)


def kernel(input_embeddings, memories_labels, memories_embeddings,
           memories_weights):
    labels = memories_labels.astype(jnp.int32)
    recip = 1.0 / jnp.maximum(jnp.arange(256, dtype=jnp.float32), 1.0)
    flat = _sc_histogram(labels, memories_weights, recip)
    return flat.reshape(B, NUM_CLASSES)


# speedup vs baseline: 26.1397x; 1.0250x over previous
import dataclasses
import functools

import jax
import jax.numpy as jnp
from jax import lax
from jax.experimental import pallas as pl
from jax.experimental.pallas import tpu as pltpu
from jax.experimental.pallas import tpu_sc as plsc

NUM_CLASSES = 1000
MIN_W = 0.1
B = 4096
M = 200
L = 16
NFULL = 12
TAIL_OFF = 184
NW = 32
RPW = B // NW
RPB = 8
NBLK = RPW // RPB
SLAB = RPB * NUM_CLASSES

_mesh = plsc.VectorSubcoreMesh(core_axis_name="c", subcore_axis_name="s")

_cp = pltpu.CompilerParams()
if "needs_layout_passes" in pltpu.CompilerParams.__dataclass_fields__:
    _cp = dataclasses.replace(_cp, needs_layout_passes=False)


@jax.jit
def _sc_histogram(labels, weights, recip):
    @functools.partial(
        pl.kernel,
        mesh=_mesh,
        compiler_params=_cp,
        out_type=jax.ShapeDtypeStruct((B, NUM_CLASSES), jnp.float32),
        scratch_types=[
            pltpu.VMEM((RPW, M), jnp.int32),
            pltpu.VMEM((RPW, M), jnp.float32),
            pltpu.VMEM((RPB, NUM_CLASSES), jnp.float32),
            pltpu.VMEM((RPB, NUM_CLASSES), jnp.float32),
            pltpu.VMEM((256,), jnp.float32),
            pltpu.SemaphoreType.DMA,
            pltpu.SemaphoreType.DMA,
        ],
    )
    def k(lab_hbm, w_hbm, recip_hbm, out_hbm, lab_v, w_v, slab0, slab1,
          recip_v, sem0, sem1):
        wid = lax.axis_index("s") * 2 + lax.axis_index("c")
        base = wid * RPW
        pltpu.sync_copy(lab_hbm.at[pl.ds(base, RPW)], lab_v)
        pltpu.sync_copy(w_hbm.at[pl.ds(base, RPW)], w_v)
        pltpu.sync_copy(recip_hbm, recip_v)

        slabs = (slab0, slab1)
        sems = (sem0, sem1)
        zeros = jnp.zeros((L,), jnp.float32)
        tail_mask = lax.iota(jnp.int32, L) >= (NFULL * L - TAIL_OFF)

        def zero_slab(slab):
            @pl.loop(0, (NUM_CLASSES // L) * L, step=4 * L)
            def _(i):
                for s in range(RPB):
                    for u in range(4):
                        slab[s, pl.ds(i + u * L, L)] = zeros
            for s in range(RPB):
                slab[s, pl.ds(NUM_CLASSES - L, L)] = zeros

        zero_slab(slab0)
        zero_slab(slab1)

        def do_row(slab, r, s):
            ws = [w_v[r, pl.ds(c * L, L)] for c in range(NFULL)]
            wt = w_v[r, pl.ds(TAIL_OFF, L)]
            masks = [w >= MIN_W for w in ws]
            mt = (wt >= MIN_W) & tail_mask
            cnt = plsc.all_reduce_population_count(mt)
            for m in masks:
                cnt = cnt + plsc.all_reduce_population_count(m)
            inv = plsc.load_gather(recip_v, [cnt])
            tiny_inv = 1e-10 * inv

            svec = jnp.full((L,), s, jnp.int32)
            for c in range(NFULL):
                lab = lab_v[r, pl.ds(c * L, L)]
                attn = ws[c] * jnp.where(masks[c], inv, tiny_inv)
                plsc.addupdate_scatter(slab, [svec, lab], attn)
            labt = lab_v[r, pl.ds(TAIL_OFF, L)]
            attnt = wt * jnp.where(mt, inv, tiny_inv)
            plsc.addupdate_scatter(slab, [svec, labt], attnt, mask=tail_mask)

        @pl.loop(0, NBLK, step=2)
        def _(rb):
            for j in range(2):
                rbx = rb + j
                slab, sem = slabs[j], sems[j]

                @pl.when(rbx >= 2)
                def _():
                    dst = out_hbm.at[pl.ds(base + (rbx - 2) * RPB, RPB)]
                    pltpu.make_async_copy(slab, dst, sem).wait()
                    zero_slab(slab)

                for s in range(RPB):
                    do_row(slab, rbx * RPB + s, s)

                dst = out_hbm.at[pl.ds(base + rbx * RPB, RPB)]
                pltpu.make_async_copy(slab, dst, sem).start()

        for j in range(2):
            rbx = NBLK - 2 + j
            dst = out_hbm.at[pl.ds(base + rbx * RPB, RPB)]
            pltpu.make_async_copy(slabs[j], dst, sems[j]).wait()

    return k(labels, weights, recip)


def kernel(input_embeddings, memories_labels, memories_embeddings,
           memories_weights):
    labels = memories_labels.astype(jnp.int32)
    recip = 1.0 / jnp.maximum(jnp.arange(256, dtype=jnp.float32), 1.0)
    return _sc_histogram(labels, memories_weights, recip)
